# Initial kernel scaffold; baseline (speedup 1.0000x reference)
import jax
import jax.numpy as jnp
from jax.experimental import pallas as pl


def kernel(tensor, weight):
    raise NotImplementedError("write your pallas kernel here")


# speedup vs baseline: 1.4636x; 1.4636x over previous
import functools

import jax
import jax.numpy as jnp
from jax import lax
from jax.experimental import pallas as pl
from jax.experimental.pallas import tpu as pltpu
from jax.experimental.pallas import tpu_sc as plsc

N_NODES = 100000
EMBED_DIM = 128

_info = plsc.get_sparse_core_info()
_NC, _NS = _info.num_cores, _info.num_subcores
_NW = _NC * _NS
_ELEMS = N_NODES * EMBED_DIM
_ELEMS_PER_W = _ELEMS // _NW
_BUF_ROWS = 125
_BUF = _BUF_ROWS * EMBED_DIM
_N_DMA = _ELEMS_PER_W // _BUF
_LANES = 16

_mesh = plsc.VectorSubcoreMesh(core_axis_name="c", subcore_axis_name="s")


@functools.partial(
    pl.kernel,
    mesh=_mesh,
    out_type=jax.ShapeDtypeStruct((_ELEMS,), jnp.float32),
    scratch_types=[
        pltpu.VMEM((_BUF,), jnp.float32),
        pltpu.SemaphoreType.DMA,
    ],
)
def _broadcast_rows(w_hbm, out_hbm, buf_v, sem):
    wid = lax.axis_index("s") * _NC + lax.axis_index("c")
    pltpu.sync_copy(w_hbm, buf_v.at[pl.ds(0, EMBED_DIM)])
    wv = [buf_v[pl.ds(d * _LANES, _LANES)] for d in range(EMBED_DIM // _LANES)]

    def _fill_row(i, _):
        for d in range(EMBED_DIM // _LANES):
            buf_v[pl.ds(i * EMBED_DIM + d * _LANES, _LANES)] = wv[d]
        return 0

    lax.fori_loop(1, _BUF_ROWS, _fill_row, 0)
    base = wid * _ELEMS_PER_W
    copies = [
        pltpu.async_copy(buf_v, out_hbm.at[pl.ds(base + j * _BUF, _BUF)], sem)
        for j in range(_N_DMA)
    ]
    for c in copies:
        c.wait()


def kernel(tensor, weight):
    del tensor
    flat = _broadcast_rows(weight.reshape(EMBED_DIM))
    return flat.reshape(N_NODES, EMBED_DIM)
